# Initial kernel scaffold; baseline (speedup 1.0000x reference)
import jax
import jax.numpy as jnp
from jax.experimental import pallas as pl


def kernel(x, neighbor_indices, importance_weights, W1, b1, W2, b2):
    raise NotImplementedError("write your pallas kernel here")


# speedup vs baseline: 1.5278x; 1.5278x over previous
import functools

import jax
import jax.numpy as jnp
from jax import lax
from jax.experimental import pallas as pl
from jax.experimental.pallas import tpu as pltpu
from jax.experimental.pallas import tpu_sc as plsc

N = 50000
D = 128
H = 64
K = 32
B = 10000

NC = 2
NS = 16
NW = NC * NS
PB = 10240
RPW = PB // NW
C = 4
CK = C * K
CH = RPW // C
LANES = 16

TILE = 2000
NT = N // TILE


def _scores_body(x_ref, w1_ref, b1_ref, w2_ref, b2_ref, o_ref):
    h = jnp.dot(x_ref[...], w1_ref[...], preferred_element_type=jnp.float32)
    h = jnp.maximum(h + b1_ref[...], 0.0)
    s = jnp.sum(h * w2_ref[...], axis=1) + b2_ref[0, 0]
    o_ref[0, 0, :] = s


def _node_scores(x, W1, b1, W2, b2):
    out = pl.pallas_call(
        _scores_body,
        grid=(NT,),
        in_specs=[
            pl.BlockSpec((TILE, D), lambda i: (i, 0)),
            pl.BlockSpec((D, H), lambda i: (0, 0)),
            pl.BlockSpec((1, H), lambda i: (0, 0)),
            pl.BlockSpec((1, H), lambda i: (0, 0)),
            pl.BlockSpec((1, 1), lambda i: (0, 0)),
        ],
        out_specs=pl.BlockSpec((1, 1, TILE), lambda i: (i, 0, 0)),
        out_shape=jax.ShapeDtypeStruct((NT, 1, TILE), jnp.float32),
    )(x, W1, b1.reshape(1, H), W2.reshape(1, H), b2.reshape(1, 1))
    return out.reshape(N)


def _bcast_lane(v, k):
    return v.at[jnp.full((LANES,), k, jnp.int32)].get(
        mode="promise_in_bounds")


def _lane_splat_reduce(v, op):
    lane = lax.iota(jnp.int32, LANES)
    for s in (1, 2, 4, 8):
        perm = jnp.bitwise_xor(lane, s)
        v = op(v, v.at[perm].get(mode="promise_in_bounds"))
    return v


def _sc_body(x_hbm, s_hbm, idx_hbm, iw_hbm, out_hbm,
             idx_v, iw_v, rows_v, scr_v, out_v, sem0, sem1):
    wid = lax.axis_index("s") * NC + lax.axis_index("c")
    pltpu.sync_copy(idx_hbm.at[wid], idx_v)
    pltpu.sync_copy(iw_hbm.at[wid], iw_v)
    sems = (sem0, sem1)

    def start(chunk, p):
        pltpu.async_copy(x_hbm.at[idx_v.at[chunk]], rows_v.at[p], sems[p])
        pltpu.async_copy(s_hbm.at[idx_v.at[chunk]], scr_v.at[p], sems[p])

    def wait(chunk, p):
        pltpu.make_async_copy(x_hbm.at[idx_v.at[chunk]], rows_v.at[p],
                              sems[p]).wait()
        pltpu.make_async_copy(s_hbm.at[idx_v.at[chunk]], scr_v.at[p],
                              sems[p]).wait()

    def compute_chunk(chunk, p):
        def row_body(bb, carry):
            base = bb * K
            l1 = scr_v[p, pl.ds(base, LANES)]
            l2 = scr_v[p, pl.ds(base + LANES, LANES)]
            m = _lane_splat_reduce(jnp.maximum(l1, l2), jnp.maximum)
            e1 = jnp.exp(l1 - m)
            e2 = jnp.exp(l2 - m)
            r = chunk * C + bb
            p1 = iw_v[r, pl.ds(0, LANES)]
            p2 = iw_v[r, pl.ds(LANES, LANES)]
            pm = _lane_splat_reduce(jnp.maximum(p1, p2), jnp.maximum)
            q1 = jnp.exp(p1 - pm)
            q2 = jnp.exp(p2 - pm)
            ae = 0.5 / _lane_splat_reduce(e1 + e2, jnp.add)
            aq = 0.5 / _lane_splat_reduce(q1 + q2, jnp.add)
            w1 = e1 * ae + q1 * aq
            w2 = e2 * ae + q2 * aq
            accs = [jnp.zeros((LANES,), jnp.float32) for _ in range(8)]
            for half, wv in ((0, w1), (1, w2)):
                for k in range(LANES):
                    wk = _bcast_lane(wv, k)
                    rowi = base + half * LANES + k
                    for dk in range(8):
                        accs[dk] = accs[dk] + wk * rows_v[p, rowi,
                                                          pl.ds(dk * LANES,
                                                                LANES)]
            for dk in range(8):
                out_v[r, pl.ds(dk * LANES, LANES)] = accs[dk]
            return carry

        lax.fori_loop(0, C, row_body, 0)

    start(0, 0)
    start(1, 1)

    def outer(c2, carry):
        for p in range(2):
            chunk = c2 * 2 + p
            wait(chunk, p)
            compute_chunk(chunk, p)

            @pl.when(chunk + 2 < CH)
            def _():
                start(chunk + 2, p)
        return carry

    lax.fori_loop(0, CH // 2, outer, 0)
    pltpu.sync_copy(out_v, out_hbm.at[pl.ds(wid * RPW, RPW)])


@functools.cache
def _pool_sc():
    return functools.partial(
        pl.kernel,
        out_type=jax.ShapeDtypeStruct((PB, D), jnp.float32),
        mesh=plsc.VectorSubcoreMesh(core_axis_name="c", subcore_axis_name="s",
                                    num_cores=NC, num_subcores=NS),
        scratch_types=[
            pltpu.VMEM((CH, CK), jnp.int32),
            pltpu.VMEM((RPW, K), jnp.float32),
            pltpu.VMEM((2, CK, D), jnp.float32),
            pltpu.VMEM((2, CK), jnp.float32),
            pltpu.VMEM((RPW, D), jnp.float32),
            pltpu.SemaphoreType.DMA,
            pltpu.SemaphoreType.DMA,
        ],
    )(_sc_body)


def kernel(x, neighbor_indices, importance_weights, W1, b1, W2, b2):
    scores = _node_scores(x, W1, b1, W2, b2)
    pad = PB - B
    idx3 = jnp.pad(neighbor_indices, ((0, pad), (0, 0))).reshape(NW, CH, CK)
    iw3 = jnp.pad(importance_weights, ((0, pad), (0, 0))).reshape(NW, RPW, K)
    out = _pool_sc()(x, scores, idx3, iw3)
    return out[:B]


# speedup vs baseline: 1.6253x; 1.0639x over previous
import functools

import jax
import jax.numpy as jnp
from jax import lax
from jax.experimental import pallas as pl
from jax.experimental.pallas import tpu as pltpu
from jax.experimental.pallas import tpu_sc as plsc

N = 50000
D = 128
H = 64
K = 32
B = 10000

NC = 2
NS = 16
NW = NC * NS
PB = 10240
R0 = 512
R1 = (PB - NS * R0) // NS
C = 4
CK = C * K
CH0 = R0 // C
CH1 = R1 // C
OG = 4
LANES = 16
NP = 50048

TILE = 2000
NT = N // TILE


def _scores_body(x_ref, w1_ref, b1_ref, w2_ref, b2_ref, o_ref):
    h = jnp.dot(x_ref[...], w1_ref[...], preferred_element_type=jnp.float32)
    h = jnp.maximum(h + b1_ref[...], 0.0)
    s = jnp.sum(h * w2_ref[...], axis=1) + b2_ref[0, 0]
    o_ref[0, 0, :] = s


def _node_scores(x, W1, b1, W2, b2):
    out = pl.pallas_call(
        _scores_body,
        grid=(NT,),
        in_specs=[
            pl.BlockSpec((TILE, D), lambda i: (i, 0)),
            pl.BlockSpec((D, H), lambda i: (0, 0)),
            pl.BlockSpec((1, H), lambda i: (0, 0)),
            pl.BlockSpec((1, H), lambda i: (0, 0)),
            pl.BlockSpec((1, 1), lambda i: (0, 0)),
        ],
        out_specs=pl.BlockSpec((1, 1, TILE), lambda i: (i, 0, 0)),
        out_shape=jax.ShapeDtypeStruct((NT, 1, TILE), jnp.float32),
    )(x, W1, b1.reshape(1, H), W2.reshape(1, H), b2.reshape(1, 1))
    return out.reshape(N)


def _bcast_lane(v, k):
    return v.at[jnp.full((LANES,), k, jnp.int32)].get(
        mode="promise_in_bounds")


def _lane_splat_reduce(v, op):
    lane = lax.iota(jnp.int32, LANES)
    for s in (1, 2, 4, 8):
        perm = jnp.bitwise_xor(lane, s)
        v = op(v, v.at[perm].get(mode="promise_in_bounds"))
    return v


def _sc_body(x_hbm, s_hbm, idx_hbm, iw_hbm, out_hbm,
             idx_v, iw_b, scores_v, rows_v, outb_v,
             sem0, sem1, iwsem0, iwsem1):
    cc = lax.axis_index("c")
    ss = lax.axis_index("s")
    on0 = cc == 0
    gbase = pl.multiple_of(jnp.where(on0, ss * R0, NS * R0 + ss * R1), 128)
    nchunks = jnp.where(on0, CH0, CH1)
    cbase = pl.multiple_of(gbase // C, 32)
    GR = OG * C
    pltpu.sync_copy(s_hbm, scores_v.at[pl.ds(0, N)])

    @pl.when(on0)
    def _():
        pltpu.sync_copy(idx_hbm.at[pl.ds(cbase, CH0)], idx_v)

    @pl.when(jnp.logical_not(on0))
    def _():
        pltpu.sync_copy(idx_hbm.at[pl.ds(cbase, CH1)], idx_v.at[pl.ds(0, CH1)])

    sems = (sem0, sem1)
    iwsems = (iwsem0, iwsem1)

    def iw_src(g):
        start_row = pl.multiple_of(gbase + g * GR, GR)
        return iw_hbm.at[pl.ds(start_row, GR)]

    def iw_fetch(g):
        for q in range(2):
            @pl.when(g % 2 == q)
            def _():
                pltpu.async_copy(iw_src(g), iw_b.at[q], iwsems[q])

    def iw_wait(g):
        for q in range(2):
            @pl.when(g % 2 == q)
            def _():
                pltpu.make_async_copy(iw_src(g), iw_b.at[q],
                                      iwsems[q]).wait()

    def start(chunk, p):
        pltpu.async_copy(x_hbm.at[idx_v.at[chunk]], rows_v.at[p], sems[p])

    def wait(chunk, p):
        pltpu.make_async_copy(x_hbm.at[idx_v.at[chunk]], rows_v.at[p],
                              sems[p]).wait()

    def compute_chunk(chunk, p):
        def row_body(bb, carry):
            base = bb * K
            i1 = idx_v[chunk, pl.ds(base, LANES)]
            i2 = idx_v[chunk, pl.ds(base + LANES, LANES)]
            l1 = plsc.load_gather(scores_v, [i1])
            l2 = plsc.load_gather(scores_v, [i2])
            m = _lane_splat_reduce(jnp.maximum(l1, l2), jnp.maximum)
            e1 = jnp.exp(l1 - m)
            e2 = jnp.exp(l2 - m)
            gq = (chunk // OG) % 2
            rg = (chunk % OG) * C + bb
            p1 = iw_b[gq, rg, pl.ds(0, LANES)]
            p2 = iw_b[gq, rg, pl.ds(LANES, LANES)]
            pm = _lane_splat_reduce(jnp.maximum(p1, p2), jnp.maximum)
            q1 = jnp.exp(p1 - pm)
            q2 = jnp.exp(p2 - pm)
            ae = 0.5 / _lane_splat_reduce(e1 + e2, jnp.add)
            aq = 0.5 / _lane_splat_reduce(q1 + q2, jnp.add)
            w1 = e1 * ae + q1 * aq
            w2 = e2 * ae + q2 * aq
            accs = [jnp.zeros((LANES,), jnp.float32) for _ in range(8)]
            for half, wv in ((0, w1), (1, w2)):
                for k in range(LANES):
                    wk = _bcast_lane(wv, k)
                    rowi = base + half * LANES + k
                    for dk in range(8):
                        accs[dk] = accs[dk] + wk * rows_v[p, rowi,
                                                          pl.ds(dk * LANES,
                                                                LANES)]
            for dk in range(8):
                outb_v[rg, pl.ds(dk * LANES, LANES)] = accs[dk]
            return carry

        lax.fori_loop(0, C, row_body, 0)

    start(0, 0)
    start(1, 1)
    iw_fetch(0)
    iw_fetch(1)

    def outer(c2, carry):
        for p in range(2):
            chunk = c2 * 2 + p
            g = chunk // OG

            @pl.when(chunk % OG == 0)
            def _():
                iw_wait(g)

            wait(chunk, p)
            compute_chunk(chunk, p)

            @pl.when(chunk + 2 < nchunks)
            def _():
                start(chunk + 2, p)

            @pl.when(chunk % OG == OG - 1)
            def _():
                ostart = pl.multiple_of(
                    gbase + (chunk - (OG - 1)) * C, OG * C)
                pltpu.sync_copy(outb_v, out_hbm.at[pl.ds(ostart, OG * C)])

                @pl.when(g + 2 < nchunks // OG)
                def _():
                    iw_fetch(g + 2)
        return carry

    lax.fori_loop(0, nchunks // 2, outer, 0)


@functools.cache
def _pool_sc():
    return functools.partial(
        pl.kernel,
        out_type=jax.ShapeDtypeStruct((PB, D), jnp.float32),
        mesh=plsc.VectorSubcoreMesh(core_axis_name="c", subcore_axis_name="s",
                                    num_cores=NC, num_subcores=NS),
        scratch_types=[
            pltpu.VMEM((CH0, CK), jnp.int32),
            pltpu.VMEM((2, OG * C, K), jnp.float32),
            pltpu.VMEM((NP,), jnp.float32),
            pltpu.VMEM((2, CK, D), jnp.float32),
            pltpu.VMEM((OG * C, D), jnp.float32),
            pltpu.SemaphoreType.DMA,
            pltpu.SemaphoreType.DMA,
            pltpu.SemaphoreType.DMA,
            pltpu.SemaphoreType.DMA,
        ],
        compiler_params=pltpu.CompilerParams(needs_layout_passes=False),
    )(_sc_body)


def kernel(x, neighbor_indices, importance_weights, W1, b1, W2, b2):
    scores = _node_scores(x, W1, b1, W2, b2)
    pad = PB - B
    idx2 = jnp.pad(neighbor_indices, ((0, pad), (0, 0))).reshape(PB // C, CK)
    iw_p = jnp.pad(importance_weights, ((0, pad), (0, 0)))
    out = _pool_sc()(x, scores, idx2, iw_p)
    return out[:B]
